# baseline (speedup 1.0000x reference)
import jax
import jax.numpy as jnp
from jax import lax
from jax.experimental import pallas as pl
from jax.experimental.pallas import tpu as pltpu
from jax.experimental.pallas import tpu_sc as plsc

B = 64
N = 20000
K = 256
LANES = 16
NUM_CORES = 2
NUM_SUBCORES = 16
NW = NUM_CORES * NUM_SUBCORES
RPW = B // NW
NV = N // LANES
CH = 512
CHV = CH // LANES
BUF = K + 2 * LANES


def _body(ct_hbm, bt_hbm, bx_hbm, p2l_hbm,
          out_ct, out_bt, out_bx, out_p2l,
          row_buf, aux_pref, box_full, bt_pref, bx_pref, buf_a, buf_b, ca_smem,
          idx_loc, ct_sel, p2l_sel, box_sel,
          sem_ct, sem_aux, sem_bt, sem_bx, *out_sems):
    wid = lax.axis_index("s") * NUM_CORES + lax.axis_index("c")

    out_cps = []
    for r in range(RPW):
        b = wid * RPW + r
        cp_ct = pltpu.async_copy(
            ct_hbm.at[b].at[pl.ds(0, CH)], row_buf.at[pl.ds(0, CH)], sem_ct)
        cp_p2l = pltpu.async_copy(
            p2l_hbm.at[b].at[pl.ds(0, CH)], aux_pref, sem_aux)
        cp_bt = pltpu.async_copy(
            bt_hbm.at[b].at[:, pl.ds(0, CH)], bt_pref, sem_bt)
        cp_bx = pltpu.async_copy(
            bx_hbm.at[b].at[:, pl.ds(0, CH)], bx_pref, sem_bx)
        cp_ct.wait()

        def scan_step(i, c, want_nonzero, buf):
            off = i * LANES
            v = row_buf[pl.ds(off, LANES)]
            m = (v > 0) if want_nonzero else (v == 0)
            ivec = lax.iota(jnp.int32, LANES) + off
            plsc.store_compressed(buf.at[pl.ds(c, LANES)], ivec, mask=m)
            return c + jnp.sum(m.astype(jnp.int32))

        def pass1(i0, c0, nv):
            def cond(st):
                i, ca = st
                return jnp.logical_and(i < nv, ca < K)

            def body(st):
                i, ca = st
                return i + 1, scan_step(i, ca, True, buf_a)

            return lax.while_loop(cond, body, (i0, c0))

        def pass2(ca, nv):
            def cond(st):
                i, cb = st
                return jnp.logical_and(i < nv, ca + cb < K)

            def body(st):
                i, cb = st
                return i + 1, scan_step(i, cb, False, buf_b)

            return lax.while_loop(cond, body, (jnp.int32(0), jnp.int32(0)))

        _, ca1 = pass1(jnp.int32(0), jnp.int32(0), CHV)
        done = ca1 >= K
        ca_smem[0] = ca1

        @pl.when(jnp.logical_not(done))
        def _():
            pltpu.sync_copy(ct_hbm.at[b], row_buf)
            _, ca2 = pass1(jnp.int32(CHV), ca1, NV)
            pass2(ca2, NV)
            ca_smem[0] = ca2

        ca = ca_smem[0]

        for i in range(K // LANES):
            j = lax.iota(jnp.int32, LANES) + (i * LANES)
            m_a = j < ca
            av = plsc.load_gather(buf_a, [j])
            bv = plsc.load_gather(buf_b, [jnp.maximum(j - ca, 0)])
            idx = jnp.where(m_a, av, bv)
            idx_loc[pl.ds(i * LANES, LANES)] = idx
            ct_sel[r, pl.ds(i * LANES, LANES)] = plsc.load_gather(
                row_buf, [idx])

        cp_p2l.wait()

        @pl.when(done)
        def _():
            for i in range(K // LANES):
                idx = idx_loc[pl.ds(i * LANES, LANES)]
                p2l_sel[r, pl.ds(i * LANES, LANES)] = plsc.load_gather(
                    aux_pref, [idx])

        @pl.when(jnp.logical_not(done))
        def _():
            pltpu.sync_copy(p2l_hbm.at[b], row_buf)
            for i in range(K // LANES):
                idx = idx_loc[pl.ds(i * LANES, LANES)]
                p2l_sel[r, pl.ds(i * LANES, LANES)] = plsc.load_gather(
                    row_buf, [idx])

        def gather_boxes(src, a):
            for i in range(K // LANES):
                idx = idx_loc[pl.ds(i * LANES, LANES)]
                for c in range(4):
                    cs = jnp.full((LANES,), c, jnp.int32)
                    box_sel[r, a, c, pl.ds(i * LANES, LANES)] = (
                        plsc.load_gather(src, [cs, idx]))

        cp_bt.wait()

        @pl.when(jnp.logical_not(done))
        def _():
            pltpu.sync_copy(bt_hbm.at[b], box_full)
            gather_boxes(box_full, 0)

        @pl.when(done)
        def _():
            gather_boxes(bt_pref, 0)

        cp_bx.wait()

        @pl.when(jnp.logical_not(done))
        def _():
            pltpu.sync_copy(bx_hbm.at[b], box_full)
            gather_boxes(box_full, 1)

        @pl.when(done)
        def _():
            gather_boxes(bx_pref, 1)

        out_cps.append(pltpu.async_copy(
            box_sel.at[r, 0], out_bt.at[b], out_sems[4 * r]))
        out_cps.append(pltpu.async_copy(
            box_sel.at[r, 1], out_bx.at[b], out_sems[4 * r + 1]))
        out_cps.append(pltpu.async_copy(
            ct_sel.at[r], out_ct.at[b], out_sems[4 * r + 2]))
        out_cps.append(pltpu.async_copy(
            p2l_sel.at[r], out_p2l.at[b], out_sems[4 * r + 3]))

    for cp in out_cps:
        cp.wait()


@jax.jit
def kernel(class_targets, box_targets, boxes, proposal_to_label_map):
    bt = box_targets.transpose(0, 2, 1)
    bx = boxes.transpose(0, 2, 1)
    run = pl.kernel(
        _body,
        out_type=[
            jax.ShapeDtypeStruct((B, K), jnp.int32),
            jax.ShapeDtypeStruct((B, 4, K), jnp.float32),
            jax.ShapeDtypeStruct((B, 4, K), jnp.float32),
            jax.ShapeDtypeStruct((B, K), jnp.int32),
        ],
        mesh=plsc.VectorSubcoreMesh(core_axis_name="c", subcore_axis_name="s"),
        compiler_params=pltpu.CompilerParams(needs_layout_passes=False),
        scratch_types=[
            pltpu.VMEM((N,), jnp.int32),
            pltpu.VMEM((CH,), jnp.int32),
            pltpu.VMEM((4, N), jnp.float32),
            pltpu.VMEM((4, CH), jnp.float32),
            pltpu.VMEM((4, CH), jnp.float32),
            pltpu.VMEM((BUF,), jnp.int32),
            pltpu.VMEM((BUF,), jnp.int32),
            pltpu.SMEM((1,), jnp.int32),
            pltpu.VMEM((K,), jnp.int32),
            pltpu.VMEM((RPW, K), jnp.int32),
            pltpu.VMEM((RPW, K), jnp.int32),
            pltpu.VMEM((RPW, 2, 4, K), jnp.float32),
        ] + [pltpu.SemaphoreType.DMA] * (4 + 4 * RPW),
    )
    out_ct, out_bt, out_bx, out_p2l = run(class_targets, bt, bx,
                                          proposal_to_label_map)
    return (out_ct,
            out_bt.transpose(0, 2, 1),
            out_bx.transpose(0, 2, 1),
            out_p2l)


# speedup vs baseline: 1.0235x; 1.0235x over previous
import jax
import jax.numpy as jnp
from jax import lax
from jax.experimental import pallas as pl
from jax.experimental.pallas import tpu as pltpu
from jax.experimental.pallas import tpu_sc as plsc

B = 64
N = 20000
K = 256
LANES = 16
NUM_CORES = 2
NUM_SUBCORES = 16
NW = NUM_CORES * NUM_SUBCORES
RPW = B // NW
NV = N // LANES
CH = 512
CHV = CH // LANES
BUF = K + 2 * LANES


def _body(ct_hbm, bt_hbm, bx_hbm, p2l_hbm,
          out_ct, out_bt, out_bx, out_p2l,
          row_buf, aux_pref, box_full, bt_pref, bx_pref, buf_a, buf_b, ca_smem,
          idx_loc, ct_sel, p2l_sel, box_sel,
          sem_ct, sem_aux, sem_bt, sem_bx):
    wid = lax.axis_index("s") * NUM_CORES + lax.axis_index("c")

    for r in range(RPW):
        b = wid * RPW + r
        cp_ct = pltpu.async_copy(
            ct_hbm.at[b].at[pl.ds(0, CH)], row_buf.at[pl.ds(0, CH)], sem_ct)
        cp_p2l = pltpu.async_copy(
            p2l_hbm.at[b].at[pl.ds(0, CH)], aux_pref, sem_aux)
        cp_bt = pltpu.async_copy(
            bt_hbm.at[b].at[:, pl.ds(0, CH)], bt_pref, sem_bt)
        cp_bx = pltpu.async_copy(
            bx_hbm.at[b].at[:, pl.ds(0, CH)], bx_pref, sem_bx)
        cp_ct.wait()

        def scan_step(i, c, want_nonzero, buf):
            off = i * LANES
            v = row_buf[pl.ds(off, LANES)]
            m = (v > 0) if want_nonzero else (v == 0)
            ivec = lax.iota(jnp.int32, LANES) + off
            plsc.store_compressed(buf.at[pl.ds(c, LANES)], ivec, mask=m)
            return c + jnp.sum(m.astype(jnp.int32))

        def pass1(i0, c0, nv):
            def cond(st):
                i, ca = st
                return jnp.logical_and(i < nv, ca < K)

            def body(st):
                i, ca = st
                return i + 1, scan_step(i, ca, True, buf_a)

            return lax.while_loop(cond, body, (i0, c0))

        def pass2(ca, nv):
            def cond(st):
                i, cb = st
                return jnp.logical_and(i < nv, ca + cb < K)

            def body(st):
                i, cb = st
                return i + 1, scan_step(i, cb, False, buf_b)

            return lax.while_loop(cond, body, (jnp.int32(0), jnp.int32(0)))

        _, ca1 = pass1(jnp.int32(0), jnp.int32(0), CHV)
        done = ca1 >= K
        ca_smem[0] = ca1

        @pl.when(jnp.logical_not(done))
        def _():
            pltpu.sync_copy(ct_hbm.at[b], row_buf)
            _, ca2 = pass1(jnp.int32(CHV), ca1, NV)
            pass2(ca2, NV)
            ca_smem[0] = ca2

        ca = ca_smem[0]

        for i in range(K // LANES):
            j = lax.iota(jnp.int32, LANES) + (i * LANES)
            m_a = j < ca
            av = plsc.load_gather(buf_a, [j])
            bv = plsc.load_gather(buf_b, [jnp.maximum(j - ca, 0)])
            idx = jnp.where(m_a, av, bv)
            idx_loc[pl.ds(i * LANES, LANES)] = idx
            ct_sel[pl.ds(i * LANES, LANES)] = plsc.load_gather(row_buf, [idx])

        cp_p2l.wait()

        @pl.when(done)
        def _():
            for i in range(K // LANES):
                idx = idx_loc[pl.ds(i * LANES, LANES)]
                p2l_sel[pl.ds(i * LANES, LANES)] = plsc.load_gather(
                    aux_pref, [idx])

        @pl.when(jnp.logical_not(done))
        def _():
            pltpu.sync_copy(p2l_hbm.at[b], row_buf)
            for i in range(K // LANES):
                idx = idx_loc[pl.ds(i * LANES, LANES)]
                p2l_sel[pl.ds(i * LANES, LANES)] = plsc.load_gather(
                    row_buf, [idx])

        def gather_boxes(src):
            for i in range(K // LANES):
                idx = idx_loc[pl.ds(i * LANES, LANES)]
                for c in range(4):
                    cs = jnp.full((LANES,), c, jnp.int32)
                    box_sel[c, pl.ds(i * LANES, LANES)] = plsc.load_gather(
                        src, [cs, idx])

        cp_bt.wait()

        @pl.when(jnp.logical_not(done))
        def _():
            pltpu.sync_copy(bt_hbm.at[b], box_full)
            gather_boxes(box_full)

        @pl.when(done)
        def _():
            gather_boxes(bt_pref)

        pltpu.sync_copy(box_sel, out_bt.at[b])

        cp_bx.wait()

        @pl.when(jnp.logical_not(done))
        def _():
            pltpu.sync_copy(bx_hbm.at[b], box_full)
            gather_boxes(box_full)

        @pl.when(done)
        def _():
            gather_boxes(bx_pref)

        pltpu.sync_copy(box_sel, out_bx.at[b])

        pltpu.sync_copy(ct_sel, out_ct.at[b])
        pltpu.sync_copy(p2l_sel, out_p2l.at[b])


@jax.jit
def kernel(class_targets, box_targets, boxes, proposal_to_label_map):
    bt = box_targets.transpose(0, 2, 1)
    bx = boxes.transpose(0, 2, 1)
    run = pl.kernel(
        _body,
        out_type=[
            jax.ShapeDtypeStruct((B, K), jnp.int32),
            jax.ShapeDtypeStruct((B, 4, K), jnp.float32),
            jax.ShapeDtypeStruct((B, 4, K), jnp.float32),
            jax.ShapeDtypeStruct((B, K), jnp.int32),
        ],
        mesh=plsc.VectorSubcoreMesh(core_axis_name="c", subcore_axis_name="s"),
        compiler_params=pltpu.CompilerParams(
            needs_layout_passes=False, disable_bounds_checks=True),
        scratch_types=[
            pltpu.VMEM((N,), jnp.int32),
            pltpu.VMEM((CH,), jnp.int32),
            pltpu.VMEM((4, N), jnp.float32),
            pltpu.VMEM((4, CH), jnp.float32),
            pltpu.VMEM((4, CH), jnp.float32),
            pltpu.VMEM((BUF,), jnp.int32),
            pltpu.VMEM((BUF,), jnp.int32),
            pltpu.SMEM((1,), jnp.int32),
            pltpu.VMEM((K,), jnp.int32),
            pltpu.VMEM((K,), jnp.int32),
            pltpu.VMEM((K,), jnp.int32),
            pltpu.VMEM((4, K), jnp.float32),
            pltpu.SemaphoreType.DMA,
            pltpu.SemaphoreType.DMA,
            pltpu.SemaphoreType.DMA,
            pltpu.SemaphoreType.DMA,
        ],
    )
    out_ct, out_bt, out_bx, out_p2l = run(class_targets, bt, bx,
                                          proposal_to_label_map)
    return (out_ct,
            out_bt.transpose(0, 2, 1),
            out_bx.transpose(0, 2, 1),
            out_p2l)
